# baseline (speedup 1.0000x reference)
import functools

import jax
import jax.numpy as jnp
from jax import lax
from jax.experimental import pallas as pl
from jax.experimental.pallas import tpu as pltpu
from jax.experimental.pallas import tpu_sc as plsc

N = 50000
E = 800000
NC = 2
NS = 16
NW = NC * NS
CH = 128
NCHUNK = 6272
EP = NCHUNK * CH
CPW = NCHUNK // NW
CPT = NCHUNK // NS
NP = 50176
TR = NP // NS
ZR = 112
NZ = 28
GW = 16
NG = 8
FW = 48
FWB = 64
LW = 128
BE = 4096
BN = 1792


def _mesh():
    return plsc.VectorSubcoreMesh(core_axis_name="c", subcore_axis_name="s")


def _make_gather(width, dtype):
    out_type = [
        jax.ShapeDtypeStruct((EP, width), dtype),
        jax.ShapeDtypeStruct((EP, width), dtype),
    ]
    scratch = [
        pltpu.VMEM((CPW, CH), jnp.int32),
        pltpu.VMEM((CPW, CH), jnp.int32),
        pltpu.VMEM((2, CH, width), dtype),
        pltpu.VMEM((2, CH, width), dtype),
    ] + [pltpu.SemaphoreType.DMA] * 8

    @functools.partial(
        pl.kernel, mesh=_mesh(), out_type=out_type, scratch_types=scratch,
        compiler_params=pltpu.CompilerParams(use_tc_tiling_on_sc=False))
    def gather_k(table, src3d, dst3d, xs, xd, sidx, didx, sbuf, dbuf, *sems):
        gs = sems[0:2]
        gd = sems[2:4]
        ws = sems[4:6]
        wd = sems[6:8]
        c = lax.axis_index("c")
        s = lax.axis_index("s")
        w = s * NC + c

        pltpu.sync_copy(src3d.at[w], sidx)
        pltpu.sync_copy(dst3d.at[w], didx)

        def g_desc(buf, idx, slot, chunk, sem):
            return pltpu.make_async_copy(
                table.at[idx.at[chunk]], buf.at[slot], sem)

        def w_desc(buf, out, slot, chunk, sem):
            base = (w * CPW + chunk) * CH
            return pltpu.make_async_copy(
                buf.at[slot], out.at[pl.ds(base, CH)], sem)

        for slot in range(2):
            g_desc(sbuf, sidx, slot, slot, gs[slot]).start()
            g_desc(dbuf, didx, slot, slot, gd[slot]).start()

        def step(m, first, last):
            for slot in range(2):
                chunk = 2 * m + slot
                g_desc(sbuf, sidx, slot, chunk, gs[slot]).wait()
                w_desc(sbuf, xs, slot, chunk, ws[slot]).start()
                g_desc(dbuf, didx, slot, chunk, gd[slot]).wait()
                w_desc(dbuf, xd, slot, chunk, wd[slot]).start()
            for slot in range(2):
                chunk = 2 * m + slot
                w_desc(sbuf, xs, slot, chunk, ws[slot]).wait()
                if not last:
                    g_desc(sbuf, sidx, slot, chunk + 2, gs[slot]).start()
                w_desc(dbuf, xd, slot, chunk, wd[slot]).wait()
                if not last:
                    g_desc(dbuf, didx, slot, chunk + 2, gd[slot]).start()
            return 0

        lax.fori_loop(0, CPW // 2 - 1, lambda m, _: step(m, False, False), 0)
        step(CPW // 2 - 1, False, True)

    return gather_k


def _make_deg():
    scratch = [
        pltpu.VMEM((CPW, CH), jnp.int32),
        pltpu.VMEM((CH, 16), jnp.float32),
        pltpu.VMEM((ZR, 16), jnp.float32),
        pltpu.VMEM_SHARED((NP, 16), jnp.float32),
    ]

    @functools.partial(
        pl.kernel, mesh=_mesh(),
        out_type=jax.ShapeDtypeStruct((NC, NP, 16), jnp.float32),
        scratch_types=scratch,
        compiler_params=pltpu.CompilerParams(use_tc_tiling_on_sc=False))
    def deg_k(dst3d, deg_hbm, didx, ones_v, zbuf, deg_sp):
        c = lax.axis_index("c")
        s = lax.axis_index("s")
        w = s * NC + c

        pltpu.sync_copy(dst3d.at[w], didx)

        def fill(i, _):
            ones_v[i] = jnp.ones((16,), jnp.float32)
            return 0
        lax.fori_loop(0, CH, fill, 0)

        def zfill(i, _):
            zbuf[i] = jnp.zeros((16,), jnp.float32)
            return 0
        lax.fori_loop(0, ZR, zfill, 0)
        for k3 in range(NZ):
            pltpu.sync_copy(zbuf, deg_sp.at[pl.ds(s * TR + k3 * ZR, ZR)])
        plsc.subcore_barrier()

        def chunk(j, _):
            pltpu.sync_copy(ones_v, deg_sp.at[didx.at[j]], add=True)
            return 0
        lax.fori_loop(0, CPW, chunk, 0)

        plsc.subcore_barrier()
        for k3 in range(NZ):
            r0 = s * TR + k3 * ZR
            pltpu.sync_copy(deg_sp.at[pl.ds(r0, ZR)],
                            deg_hbm.at[c, pl.ds(r0, ZR)])

    return deg_k


NB = 8


def _make_scatter():
    scratch = [
        pltpu.VMEM((CPT, CH), jnp.int32),
        pltpu.VMEM((NB, CH, GW), jnp.float32),
        pltpu.VMEM((ZR, GW), jnp.float32),
        pltpu.VMEM_SHARED((NP, GW), jnp.float32),
    ] + [pltpu.SemaphoreType.DMA] * (2 * NB)

    @functools.partial(
        pl.kernel, mesh=_mesh(),
        out_type=jax.ShapeDtypeStruct((NP, LW), jnp.float32),
        scratch_types=scratch,
        compiler_params=pltpu.CompilerParams(use_tc_tiling_on_sc=False))
    def scatter_k(e_hbm, dst3d, agg_hbm, didx, ebuf, zbuf, acc_sp, *sems):
        rs = sems[0:NB]
        ss = sems[NB:2 * NB]
        c = lax.axis_index("c")
        s = lax.axis_index("s")

        for h in range(CPT // CPW):
            pltpu.sync_copy(dst3d.at[(CPT // CPW) * s + h],
                            didx.at[pl.ds(h * CPW, CPW)])

        def zfill(i, _):
            zbuf[i] = jnp.zeros((16,), jnp.float32)
            return 0
        lax.fori_loop(0, ZR, zfill, 0)

        for gi in range(NG // NC):
            g = (NG // NC) * c + gi
            for k3 in range(NZ):
                pltpu.sync_copy(
                    zbuf, acc_sp.at[pl.ds(s * TR + k3 * ZR, ZR)])
            plsc.subcore_barrier()

            def r_desc(b, chunk):
                base = (s * CPT + chunk) * CH
                return pltpu.make_async_copy(
                    e_hbm.at[pl.ds(base, CH), pl.ds(g * GW, GW)],
                    ebuf.at[b], rs[b])

            def s_desc(b, chunk):
                return pltpu.make_async_copy(
                    ebuf.at[b], acc_sp.at[didx.at[chunk]], ss[b])

            for b in range(NB):
                r_desc(b, b).start()

            def round_(k, last):
                for b in range(NB):
                    chunk = k * NB + b
                    r_desc(b, chunk).wait()
                    s_desc(b, chunk).start(add=True)
                for b in range(NB):
                    chunk = k * NB + b
                    s_desc(b, chunk).wait()
                    if not last:
                        r_desc(b, chunk + NB).start()
                return 0

            lax.fori_loop(0, CPT // NB - 1,
                          lambda k, _: round_(k, False), 0)
            round_(CPT // NB - 1, True)
            plsc.subcore_barrier()

            for k3 in range(NZ):
                r0 = s * TR + k3 * ZR
                pltpu.sync_copy(acc_sp.at[pl.ds(r0, ZR)],
                                agg_hbm.at[pl.ds(r0, ZR),
                                           pl.ds(g * GW, GW)])
            plsc.subcore_barrier()

    return scatter_k


_DOT = functools.partial(jnp.dot, preferred_element_type=jnp.float32,
                         precision=jax.lax.Precision.DEFAULT)


def _edge1_body(xs, xd, gpbd, wxs, wxd, wg, b1, out):
    m = _DOT(xs[...], wxs[...]) + _DOT(xd[...], wxd[...])
    m = m + _DOT(gpbd[...], wg[...])
    out[...] = jnp.maximum(m + b1[...], 0.0)


def _edge2_body(hs, hd, e1, w2s, w2d, w2e, b2, out):
    m = _DOT(hs[...], w2s[...]) + _DOT(hd[...], w2d[...])
    m = m + _DOT(e1[...], w2e[...])
    out[...] = jnp.maximum(m + b2[...], 0.0)


def _node1_body(feat, agg, deg, wnx, wna, bn, out):
    d = jnp.maximum(deg[0, :, 0:1] + deg[1, :, 0:1], 1.0)
    aggn = agg[...] / d
    h = jnp.maximum(_DOT(feat[...], wnx[...]) + _DOT(aggn, wna[...])
                    + bn[...], 0.0)
    out[...] = h.astype(jnp.bfloat16)


def _node2_body(h1, agg, deg, tan, t, wnx, wna, bn, wh, bh, out):
    d = jnp.maximum(deg[0, :, 0:1] + deg[1, :, 0:1], 1.0)
    aggn = agg[...] / d
    h2 = jnp.maximum(_DOT(h1[...], wnx[...]) + _DOT(aggn, wna[...])
                     + bn[...], 0.0)
    head = _DOT(h2, wh[...]) + bh[...]
    eh = jnp.exp(-jnp.abs(head[:, 2:3]) / t[...])
    bh_ = jnp.abs(head[:, 3:4])
    sg = jnp.abs(head[:, 4:5])
    o3 = (head[:, 0:1] * tan[:, 0:3] + head[:, 1:2] * tan[:, 3:6]) / bh_ * eh
    out[...] = jnp.concatenate(
        [o3, sg, jnp.zeros((o3.shape[0], 4), jnp.float32)], axis=1)


def _full(shape):
    return pl.BlockSpec(shape, lambda i: tuple(0 for _ in shape))


def _edge1_call(xs, xd, gpbd, wxs, wxd, wg, b1):
    return pl.pallas_call(
        _edge1_body,
        grid=(EP // BE,),
        in_specs=[
            pl.BlockSpec((BE, FWB), lambda i: (i, 0)),
            pl.BlockSpec((BE, FWB), lambda i: (i, 0)),
            pl.BlockSpec((BE, 32), lambda i: (i, 0)),
            _full((FWB, LW)), _full((FWB, LW)), _full((32, LW)),
            _full((1, LW)),
        ],
        out_specs=pl.BlockSpec((BE, LW), lambda i: (i, 0)),
        out_shape=jax.ShapeDtypeStruct((EP, LW), jnp.float32),
    )(xs, xd, gpbd, wxs, wxd, wg, b1)


def _edge2_call(hs, hd, e1, w2s, w2d, w2e, b2):
    return pl.pallas_call(
        _edge2_body,
        grid=(EP // BE,),
        in_specs=[
            pl.BlockSpec((BE, LW), lambda i: (i, 0)),
            pl.BlockSpec((BE, LW), lambda i: (i, 0)),
            pl.BlockSpec((BE, LW), lambda i: (i, 0)),
            _full((LW, LW)), _full((LW, LW)), _full((LW, LW)),
            _full((1, LW)),
        ],
        out_specs=pl.BlockSpec((BE, LW), lambda i: (i, 0)),
        out_shape=jax.ShapeDtypeStruct((EP, LW), jnp.float32),
    )(hs, hd, e1, w2s, w2d, w2e, b2)


def _node1_call(feat, agg, deg, wnx, wna, bn):
    return pl.pallas_call(
        _node1_body,
        grid=(NP // BN,),
        in_specs=[
            pl.BlockSpec((BN, FWB), lambda i: (i, 0)),
            pl.BlockSpec((BN, LW), lambda i: (i, 0)),
            pl.BlockSpec((2, BN, 16), lambda i: (0, i, 0)),
            _full((FWB, LW)), _full((LW, LW)), _full((1, LW)),
        ],
        out_specs=pl.BlockSpec((BN, LW), lambda i: (i, 0)),
        out_shape=jax.ShapeDtypeStruct((NP, LW), jnp.bfloat16),
    )(feat, agg, deg, wnx, wna, bn)


def _node2_call(h1, agg, deg, tan6, t, wnx, wna, bn, wh, bh):
    return pl.pallas_call(
        _node2_body,
        grid=(NP // BN,),
        in_specs=[
            pl.BlockSpec((BN, LW), lambda i: (i, 0)),
            pl.BlockSpec((BN, LW), lambda i: (i, 0)),
            pl.BlockSpec((2, BN, 16), lambda i: (0, i, 0)),
            pl.BlockSpec((BN, 6), lambda i: (i, 0)),
            pl.BlockSpec((BN, 1), lambda i: (i, 0)),
            _full((LW, LW)), _full((LW, LW)), _full((1, LW)),
            _full((LW, 8)), _full((1, 8)),
        ],
        out_specs=pl.BlockSpec((BN, 8), lambda i: (i, 0)),
        out_shape=jax.ShapeDtypeStruct((NP, 8), jnp.float32),
    )(h1, agg, deg, tan6, t, wnx, wna, bn, wh, bh)


def _pack_weights(params):
    z = jnp.zeros
    f32 = jnp.float32
    stacks1 = ["conv1", "Econv1", "Bconv1", "sigconv1"]
    xw = [32, 32, 33, 33]

    w1xs = z((FW, LW), f32)
    w1xd = z((FW, LW), f32)
    w1g = z((32, LW), f32)
    b1 = []
    wn1x = z((FW, LW), f32)
    wn1a = z((LW, LW), f32)
    bn1 = []
    for s4, name in enumerate(stacks1):
        ew = params[name]["edge"]["W"]
        nw = params[name]["node"]["W"]
        k = xw[s4]
        c0 = 32 * s4
        w1xs = w1xs.at[0:k, c0:c0 + 32].set(ew[0:k])
        w1xd = w1xd.at[0:k, c0:c0 + 32].set(ew[k:2 * k])
        if name == "Bconv1":
            w1g = w1g.at[16:17, c0:c0 + 32].set(ew[2 * k:2 * k + 1])
        else:
            w1g = w1g.at[0:16, c0:c0 + 32].set(ew[2 * k:2 * k + 16])
        b1.append(params[name]["edge"]["b"])
        wn1x = wn1x.at[0:k, c0:c0 + 32].set(nw[0:k])
        wn1a = wn1a.at[c0:c0 + 32, c0:c0 + 32].set(nw[k:k + 32])
        bn1.append(params[name]["node"]["b"])
    b1 = jnp.concatenate(b1).reshape(1, LW)
    bn1 = jnp.concatenate(bn1).reshape(1, LW)

    w2s = z((LW, LW), f32)
    w2d = z((LW, LW), f32)
    w2e = z((LW, LW), f32)
    b2 = []
    wn2x = z((LW, LW), f32)
    wn2a = z((LW, LW), f32)
    bn2 = []
    for s4, name in enumerate(["conv2", "Econv2", "Bconv2", "sigconv2"]):
        ew = params[name]["edge"]["W"]
        nw = params[name]["node"]["W"]
        c0 = 32 * s4
        w2s = w2s.at[c0:c0 + 32, c0:c0 + 32].set(ew[0:32])
        w2d = w2d.at[c0:c0 + 32, c0:c0 + 32].set(ew[32:64])
        w2e = w2e.at[c0:c0 + 32, c0:c0 + 32].set(ew[64:96])
        b2.append(params[name]["edge"]["b"])
        wn2x = wn2x.at[c0:c0 + 32, c0:c0 + 32].set(nw[0:32])
        wn2a = wn2a.at[c0:c0 + 32, c0:c0 + 32].set(nw[32:64])
        bn2.append(params[name]["node"]["b"])
    b2 = jnp.concatenate(b2).reshape(1, LW)
    bn2 = jnp.concatenate(bn2).reshape(1, LW)

    wh = z((LW, 8), f32)
    wh = wh.at[0:32, 0:2].set(params["linear"]["W"])
    wh = wh.at[32:64, 2:3].set(params["Elinear"]["W"])
    wh = wh.at[64:96, 3:4].set(params["Blinear"]["W"])
    wh = wh.at[96:128, 4:5].set(params["siglinear"]["W"])
    bh = z((1, 8), f32)
    bh = bh.at[0, 0:2].set(params["linear"]["b"])
    bh = bh.at[0, 2].set(params["Elinear"]["b"][0])
    bh = bh.at[0, 3].set(params["Blinear"]["b"][0])
    bh = bh.at[0, 4].set(params["siglinear"]["b"][0])

    return dict(w1xs=w1xs, w1xd=w1xd, w1g=w1g, b1=b1,
                wn1x=wn1x, wn1a=wn1a, bn1=bn1,
                w2s=w2s, w2d=w2d, w2e=w2e, b2=b2,
                wn2x=wn2x, wn2a=wn2a, bn2=bn2, wh=wh, bh=bh)


_make_gather = functools.cache(_make_gather)
_make_deg = functools.cache(_make_deg)
_make_scatter = functools.cache(_make_scatter)


def kernel(edge_index, feature_GP, feature_Node, feature_bdott, feature_tan,
           feature_T, params):
    f32 = jnp.float32
    src = edge_index[0]
    dst = edge_index[1]
    pad = EP - E
    src2d = jnp.concatenate(
        [src, jnp.zeros((pad,), jnp.int32)]).reshape(NCHUNK, CH)
    dst2d = jnp.concatenate(
        [dst, jnp.full((pad,), N, jnp.int32)]).reshape(NCHUNK, CH)
    featp = jnp.concatenate(
        [feature_Node, feature_T, jnp.zeros((N, FW - 33), f32)], axis=1)
    featp = jnp.concatenate([featp, jnp.zeros((NP - N, FW), f32)], axis=0)
    featp = jnp.concatenate(
        [featp, jnp.zeros((NP, FWB - FW), f32)], axis=1).astype(jnp.bfloat16)
    gpbd = jnp.concatenate(
        [feature_GP, feature_bdott, jnp.zeros((E, 15), f32)], axis=1)
    gpbd = jnp.concatenate(
        [gpbd, jnp.zeros((pad, 32), f32)], axis=0).astype(jnp.bfloat16)
    tan6 = jnp.concatenate(
        [feature_tan.reshape(N, 6), jnp.zeros((NP - N, 6), f32)], axis=0)
    tp = jnp.concatenate([feature_T, jnp.ones((NP - N, 1), f32)], axis=0)
    w = _pack_weights(params)
    w1xs = jnp.zeros((FWB, LW), f32).at[0:FW].set(w["w1xs"])
    w1xd = jnp.zeros((FWB, LW), f32).at[0:FW].set(w["w1xd"])
    wn1x = jnp.zeros((FWB, LW), f32).at[0:FW].set(w["wn1x"])

    _gather48 = _make_gather(FWB, jnp.bfloat16)
    _gather128 = _make_gather(LW, jnp.bfloat16)
    _scatter = _make_scatter()

    src3d = src2d.reshape(NW, CPW, CH)
    dst3d = dst2d.reshape(NW, CPW, CH)

    xs, xd = _gather48(featp, src3d, dst3d)
    deg = _make_deg()(dst3d)
    bf = jnp.bfloat16
    e1 = _edge1_call(xs, xd, gpbd, w1xs.astype(bf), w1xd.astype(bf),
                     w["w1g"].astype(bf), w["b1"])
    agg1 = _scatter(e1, dst3d)
    h1 = _node1_call(featp, agg1, deg, wn1x.astype(bf), w["wn1a"], w["bn1"])
    hs, hd = _gather128(h1, src3d, dst3d)
    e2 = _edge2_call(hs, hd, e1, w["w2s"].astype(bf), w["w2d"].astype(bf),
                     w["w2e"], w["b2"])
    agg2 = _scatter(e2, dst3d)
    o8 = _node2_call(h1, agg2, deg, tan6, tp,
                     w["wn2x"].astype(bf), w["wn2a"], w["bn2"], w["wh"],
                     w["bh"])
    return (o8[:N, 0:3], o8[:N, 3:4])


# speedup vs baseline: 1.0307x; 1.0307x over previous
import functools

import jax
import jax.numpy as jnp
from jax import lax
from jax.experimental import pallas as pl
from jax.experimental.pallas import tpu as pltpu
from jax.experimental.pallas import tpu_sc as plsc

N = 50000
E = 800000
NC = 2
NS = 16
NW = NC * NS
CH = 128
NCHUNK = 6272
EP = NCHUNK * CH
CPW = NCHUNK // NW
CPT = NCHUNK // NS
NP = 50176
TR = NP // NS
ZR = 112
NZ = 28
GW = 16
NG = 8
FW = 48
FWB = 64
LW = 128
BE = 4096
BN = 1792


def _mesh():
    return plsc.VectorSubcoreMesh(core_axis_name="c", subcore_axis_name="s")


def _make_gather(width, dtype):
    out_type = [
        jax.ShapeDtypeStruct((EP, width), dtype),
        jax.ShapeDtypeStruct((EP, width), dtype),
    ]
    scratch = [
        pltpu.VMEM((CPW, CH), jnp.int32),
        pltpu.VMEM((CPW, CH), jnp.int32),
        pltpu.VMEM((2, CH, width), dtype),
        pltpu.VMEM((2, CH, width), dtype),
    ] + [pltpu.SemaphoreType.DMA] * 8

    @functools.partial(
        pl.kernel, mesh=_mesh(), out_type=out_type, scratch_types=scratch,
        compiler_params=pltpu.CompilerParams(use_tc_tiling_on_sc=False))
    def gather_k(table, src3d, dst3d, xs, xd, sidx, didx, sbuf, dbuf, *sems):
        gs = sems[0:2]
        gd = sems[2:4]
        ws = sems[4:6]
        wd = sems[6:8]
        c = lax.axis_index("c")
        s = lax.axis_index("s")
        w = s * NC + c

        pltpu.sync_copy(src3d.at[w], sidx)
        pltpu.sync_copy(dst3d.at[w], didx)

        def g_desc(buf, idx, slot, chunk, sem):
            return pltpu.make_async_copy(
                table.at[idx.at[chunk]], buf.at[slot], sem)

        def w_desc(buf, out, slot, chunk, sem):
            base = (w * CPW + chunk) * CH
            return pltpu.make_async_copy(
                buf.at[slot], out.at[pl.ds(base, CH)], sem)

        for slot in range(2):
            g_desc(sbuf, sidx, slot, slot, gs[slot]).start()
            g_desc(dbuf, didx, slot, slot, gd[slot]).start()

        def step(m, first, last):
            for slot in range(2):
                chunk = 2 * m + slot
                g_desc(sbuf, sidx, slot, chunk, gs[slot]).wait()
                w_desc(sbuf, xs, slot, chunk, ws[slot]).start()
                g_desc(dbuf, didx, slot, chunk, gd[slot]).wait()
                w_desc(dbuf, xd, slot, chunk, wd[slot]).start()
            for slot in range(2):
                chunk = 2 * m + slot
                w_desc(sbuf, xs, slot, chunk, ws[slot]).wait()
                if not last:
                    g_desc(sbuf, sidx, slot, chunk + 2, gs[slot]).start()
                w_desc(dbuf, xd, slot, chunk, wd[slot]).wait()
                if not last:
                    g_desc(dbuf, didx, slot, chunk + 2, gd[slot]).start()
            return 0

        lax.fori_loop(0, CPW // 2 - 1, lambda m, _: step(m, False, False), 0)
        step(CPW // 2 - 1, False, True)

    return gather_k


def _make_deg():
    scratch = [
        pltpu.VMEM((CPW, CH), jnp.int32),
        pltpu.VMEM((CH, 16), jnp.float32),
        pltpu.VMEM((ZR, 16), jnp.float32),
        pltpu.VMEM_SHARED((NP, 16), jnp.float32),
    ]

    @functools.partial(
        pl.kernel, mesh=_mesh(),
        out_type=jax.ShapeDtypeStruct((NC, NP, 16), jnp.float32),
        scratch_types=scratch,
        compiler_params=pltpu.CompilerParams(use_tc_tiling_on_sc=False))
    def deg_k(dst3d, deg_hbm, didx, ones_v, zbuf, deg_sp):
        c = lax.axis_index("c")
        s = lax.axis_index("s")
        w = s * NC + c

        pltpu.sync_copy(dst3d.at[w], didx)

        def fill(i, _):
            ones_v[i] = jnp.ones((16,), jnp.float32)
            return 0
        lax.fori_loop(0, CH, fill, 0)

        def zfill(i, _):
            zbuf[i] = jnp.zeros((16,), jnp.float32)
            return 0
        lax.fori_loop(0, ZR, zfill, 0)
        for k3 in range(NZ):
            pltpu.sync_copy(zbuf, deg_sp.at[pl.ds(s * TR + k3 * ZR, ZR)])
        plsc.subcore_barrier()

        def chunk(j, _):
            pltpu.sync_copy(ones_v, deg_sp.at[didx.at[j]], add=True)
            return 0
        lax.fori_loop(0, CPW, chunk, 0)

        plsc.subcore_barrier()
        for k3 in range(NZ):
            r0 = s * TR + k3 * ZR
            pltpu.sync_copy(deg_sp.at[pl.ds(r0, ZR)],
                            deg_hbm.at[c, pl.ds(r0, ZR)])

    return deg_k


NB = 8


def _make_scatter():
    scratch = [
        pltpu.VMEM((CPT, CH), jnp.int32),
        pltpu.VMEM((NB, CH, GW), jnp.float32),
        pltpu.VMEM((ZR, GW), jnp.float32),
        pltpu.VMEM_SHARED((NP, GW), jnp.float32),
    ] + [pltpu.SemaphoreType.DMA] * (2 * NB)

    @functools.partial(
        pl.kernel, mesh=_mesh(),
        out_type=jax.ShapeDtypeStruct((NP, LW), jnp.float32),
        scratch_types=scratch,
        compiler_params=pltpu.CompilerParams(use_tc_tiling_on_sc=False))
    def scatter_k(e_hbm, dst3d, agg_hbm, didx, ebuf, zbuf, acc_sp, *sems):
        rs = sems[0:NB]
        ss = sems[NB:2 * NB]
        c = lax.axis_index("c")
        s = lax.axis_index("s")

        for h in range(CPT // CPW):
            pltpu.sync_copy(dst3d.at[(CPT // CPW) * s + h],
                            didx.at[pl.ds(h * CPW, CPW)])

        def zfill(i, _):
            zbuf[i] = jnp.zeros((16,), jnp.float32)
            return 0
        lax.fori_loop(0, ZR, zfill, 0)

        for gi in range(NG // NC):
            g = (NG // NC) * c + gi
            for k3 in range(NZ):
                pltpu.sync_copy(
                    zbuf, acc_sp.at[pl.ds(s * TR + k3 * ZR, ZR)])
            plsc.subcore_barrier()

            def r_desc(b, chunk):
                base = (s * CPT + chunk) * CH
                return pltpu.make_async_copy(
                    e_hbm.at[pl.ds(base, CH), pl.ds(g * GW, GW)],
                    ebuf.at[b], rs[b])

            def s_desc(b, chunk):
                return pltpu.make_async_copy(
                    ebuf.at[b], acc_sp.at[didx.at[chunk]], ss[b])

            for b in range(NB):
                r_desc(b, b).start()

            def round_(k, last):
                for b in range(NB):
                    chunk = k * NB + b
                    r_desc(b, chunk).wait()
                    s_desc(b, chunk).start(add=True)
                for b in range(NB):
                    chunk = k * NB + b
                    s_desc(b, chunk).wait()
                    if not last:
                        r_desc(b, chunk + NB).start()
                return 0

            lax.fori_loop(0, CPT // NB - 1,
                          lambda k, _: round_(k, False), 0)
            round_(CPT // NB - 1, True)
            plsc.subcore_barrier()

            for k3 in range(NZ):
                r0 = s * TR + k3 * ZR
                pltpu.sync_copy(acc_sp.at[pl.ds(r0, ZR)],
                                agg_hbm.at[pl.ds(r0, ZR),
                                           pl.ds(g * GW, GW)])
            plsc.subcore_barrier()

    return scatter_k


_DOT = functools.partial(jnp.dot, preferred_element_type=jnp.float32,
                         precision=jax.lax.Precision.DEFAULT)


def _edge1_body(xs, xd, gpbd, wxs, wxd, wg, b1, out):
    m = _DOT(xs[...], wxs[...]) + _DOT(xd[...], wxd[...])
    m = m + _DOT(gpbd[...], wg[...])
    out[...] = jnp.maximum(m + b1[...], 0.0)


def _edge2_body(hs, hd, e1, w2s, w2d, w2e, b2, out):
    m = _DOT(hs[...], w2s[...]) + _DOT(hd[...], w2d[...])
    m = m + _DOT(e1[...], w2e[...])
    out[...] = jnp.maximum(m + b2[...], 0.0)


def _node1_body(feat, agg, deg, wnx, wna, bn, out):
    d = jnp.maximum(deg[0, :, 0:1] + deg[1, :, 0:1], 1.0)
    aggn = agg[...] / d
    h = jnp.maximum(_DOT(feat[...], wnx[...]) + _DOT(aggn, wna[...])
                    + bn[...], 0.0)
    out[...] = h.astype(jnp.bfloat16)


def _node2_body(h1, agg, deg, tan, t, wnx, wna, bn, wh, bh, out):
    d = jnp.maximum(deg[0, :, 0:1] + deg[1, :, 0:1], 1.0)
    aggn = agg[...] / d
    h2 = jnp.maximum(_DOT(h1[...], wnx[...]) + _DOT(aggn, wna[...])
                     + bn[...], 0.0)
    head = _DOT(h2, wh[...]) + bh[...]
    eh = jnp.exp(-jnp.abs(head[:, 2:3]) / t[...])
    bh_ = jnp.abs(head[:, 3:4])
    sg = jnp.abs(head[:, 4:5])
    o3 = (head[:, 0:1] * tan[:, 0:3] + head[:, 1:2] * tan[:, 3:6]) / bh_ * eh
    out[...] = jnp.concatenate(
        [o3, sg, jnp.zeros((o3.shape[0], 4), jnp.float32)], axis=1)


def _full(shape):
    return pl.BlockSpec(shape, lambda i: tuple(0 for _ in shape))


def _edge1_call(xs, xd, gpbd, wxs, wxd, wg, b1):
    return pl.pallas_call(
        _edge1_body,
        grid=(EP // BE,),
        in_specs=[
            pl.BlockSpec((BE, FW), lambda i: (i, 0)),
            pl.BlockSpec((BE, FW), lambda i: (i, 0)),
            pl.BlockSpec((BE, 32), lambda i: (i, 0)),
            _full((FW, LW)), _full((FW, LW)), _full((32, LW)),
            _full((1, LW)),
        ],
        out_specs=pl.BlockSpec((BE, LW), lambda i: (i, 0)),
        out_shape=jax.ShapeDtypeStruct((EP, LW), jnp.float32),
    )(xs, xd, gpbd, wxs, wxd, wg, b1)


def _edge2_call(hs, hd, e1, w2s, w2d, w2e, b2):
    return pl.pallas_call(
        _edge2_body,
        grid=(EP // BE,),
        in_specs=[
            pl.BlockSpec((BE, LW), lambda i: (i, 0)),
            pl.BlockSpec((BE, LW), lambda i: (i, 0)),
            pl.BlockSpec((BE, LW), lambda i: (i, 0)),
            _full((LW, LW)), _full((LW, LW)), _full((LW, LW)),
            _full((1, LW)),
        ],
        out_specs=pl.BlockSpec((BE, LW), lambda i: (i, 0)),
        out_shape=jax.ShapeDtypeStruct((EP, LW), jnp.float32),
    )(hs, hd, e1, w2s, w2d, w2e, b2)


def _node1_call(feat, agg, deg, wnx, wna, bn):
    return pl.pallas_call(
        _node1_body,
        grid=(NP // BN,),
        in_specs=[
            pl.BlockSpec((BN, FW), lambda i: (i, 0)),
            pl.BlockSpec((BN, LW), lambda i: (i, 0)),
            pl.BlockSpec((2, BN, 16), lambda i: (0, i, 0)),
            _full((FW, LW)), _full((LW, LW)), _full((1, LW)),
        ],
        out_specs=pl.BlockSpec((BN, LW), lambda i: (i, 0)),
        out_shape=jax.ShapeDtypeStruct((NP, LW), jnp.bfloat16),
    )(feat, agg, deg, wnx, wna, bn)


def _node2_call(h1, agg, deg, tan6, t, wnx, wna, bn, wh, bh):
    return pl.pallas_call(
        _node2_body,
        grid=(NP // BN,),
        in_specs=[
            pl.BlockSpec((BN, LW), lambda i: (i, 0)),
            pl.BlockSpec((BN, LW), lambda i: (i, 0)),
            pl.BlockSpec((2, BN, 16), lambda i: (0, i, 0)),
            pl.BlockSpec((BN, 6), lambda i: (i, 0)),
            pl.BlockSpec((BN, 1), lambda i: (i, 0)),
            _full((LW, LW)), _full((LW, LW)), _full((1, LW)),
            _full((LW, 8)), _full((1, 8)),
        ],
        out_specs=pl.BlockSpec((BN, 8), lambda i: (i, 0)),
        out_shape=jax.ShapeDtypeStruct((NP, 8), jnp.float32),
    )(h1, agg, deg, tan6, t, wnx, wna, bn, wh, bh)


def _pack_weights(params):
    z = jnp.zeros
    f32 = jnp.float32
    stacks1 = ["conv1", "Econv1", "Bconv1", "sigconv1"]
    xw = [32, 32, 33, 33]

    w1xs = z((FW, LW), f32)
    w1xd = z((FW, LW), f32)
    w1g = z((32, LW), f32)
    b1 = []
    wn1x = z((FW, LW), f32)
    wn1a = z((LW, LW), f32)
    bn1 = []
    for s4, name in enumerate(stacks1):
        ew = params[name]["edge"]["W"]
        nw = params[name]["node"]["W"]
        k = xw[s4]
        c0 = 32 * s4
        w1xs = w1xs.at[0:k, c0:c0 + 32].set(ew[0:k])
        w1xd = w1xd.at[0:k, c0:c0 + 32].set(ew[k:2 * k])
        if name == "Bconv1":
            w1g = w1g.at[16:17, c0:c0 + 32].set(ew[2 * k:2 * k + 1])
        else:
            w1g = w1g.at[0:16, c0:c0 + 32].set(ew[2 * k:2 * k + 16])
        b1.append(params[name]["edge"]["b"])
        wn1x = wn1x.at[0:k, c0:c0 + 32].set(nw[0:k])
        wn1a = wn1a.at[c0:c0 + 32, c0:c0 + 32].set(nw[k:k + 32])
        bn1.append(params[name]["node"]["b"])
    b1 = jnp.concatenate(b1).reshape(1, LW)
    bn1 = jnp.concatenate(bn1).reshape(1, LW)

    w2s = z((LW, LW), f32)
    w2d = z((LW, LW), f32)
    w2e = z((LW, LW), f32)
    b2 = []
    wn2x = z((LW, LW), f32)
    wn2a = z((LW, LW), f32)
    bn2 = []
    for s4, name in enumerate(["conv2", "Econv2", "Bconv2", "sigconv2"]):
        ew = params[name]["edge"]["W"]
        nw = params[name]["node"]["W"]
        c0 = 32 * s4
        w2s = w2s.at[c0:c0 + 32, c0:c0 + 32].set(ew[0:32])
        w2d = w2d.at[c0:c0 + 32, c0:c0 + 32].set(ew[32:64])
        w2e = w2e.at[c0:c0 + 32, c0:c0 + 32].set(ew[64:96])
        b2.append(params[name]["edge"]["b"])
        wn2x = wn2x.at[c0:c0 + 32, c0:c0 + 32].set(nw[0:32])
        wn2a = wn2a.at[c0:c0 + 32, c0:c0 + 32].set(nw[32:64])
        bn2.append(params[name]["node"]["b"])
    b2 = jnp.concatenate(b2).reshape(1, LW)
    bn2 = jnp.concatenate(bn2).reshape(1, LW)

    wh = z((LW, 8), f32)
    wh = wh.at[0:32, 0:2].set(params["linear"]["W"])
    wh = wh.at[32:64, 2:3].set(params["Elinear"]["W"])
    wh = wh.at[64:96, 3:4].set(params["Blinear"]["W"])
    wh = wh.at[96:128, 4:5].set(params["siglinear"]["W"])
    bh = z((1, 8), f32)
    bh = bh.at[0, 0:2].set(params["linear"]["b"])
    bh = bh.at[0, 2].set(params["Elinear"]["b"][0])
    bh = bh.at[0, 3].set(params["Blinear"]["b"][0])
    bh = bh.at[0, 4].set(params["siglinear"]["b"][0])

    return dict(w1xs=w1xs, w1xd=w1xd, w1g=w1g, b1=b1,
                wn1x=wn1x, wn1a=wn1a, bn1=bn1,
                w2s=w2s, w2d=w2d, w2e=w2e, b2=b2,
                wn2x=wn2x, wn2a=wn2a, bn2=bn2, wh=wh, bh=bh)


_make_gather = functools.cache(_make_gather)
_make_deg = functools.cache(_make_deg)
_make_scatter = functools.cache(_make_scatter)


def kernel(edge_index, feature_GP, feature_Node, feature_bdott, feature_tan,
           feature_T, params):
    f32 = jnp.float32
    src = edge_index[0]
    dst = edge_index[1]
    pad = EP - E
    src2d = jnp.concatenate(
        [src, jnp.zeros((pad,), jnp.int32)]).reshape(NCHUNK, CH)
    dst2d = jnp.concatenate(
        [dst, jnp.full((pad,), N, jnp.int32)]).reshape(NCHUNK, CH)
    featp = jnp.concatenate(
        [feature_Node, feature_T, jnp.zeros((N, FW - 33), f32)], axis=1)
    featp = jnp.concatenate([featp, jnp.zeros((NP - N, FW), f32)], axis=0)
    gpbd = jnp.concatenate(
        [feature_GP, feature_bdott, jnp.zeros((E, 15), f32)], axis=1)
    gpbd = jnp.concatenate([gpbd, jnp.zeros((pad, 32), f32)], axis=0)
    tan6 = jnp.concatenate(
        [feature_tan.reshape(N, 6), jnp.zeros((NP - N, 6), f32)], axis=0)
    tp = jnp.concatenate([feature_T, jnp.ones((NP - N, 1), f32)], axis=0)
    w = _pack_weights(params)


    _gather48 = _make_gather(FW, jnp.float32)
    _gather128 = _make_gather(LW, jnp.bfloat16)
    _scatter = _make_scatter()

    src3d = src2d.reshape(NW, CPW, CH)
    dst3d = dst2d.reshape(NW, CPW, CH)

    xs, xd = _gather48(featp, src3d, dst3d)
    deg = _make_deg()(dst3d)
    bf = jnp.bfloat16
    e1 = _edge1_call(xs, xd, gpbd, w["w1xs"], w["w1xd"], w["w1g"], w["b1"])
    agg1 = _scatter(e1, dst3d)
    h1 = _node1_call(featp, agg1, deg, w["wn1x"], w["wn1a"], w["bn1"])
    hs, hd = _gather128(h1, src3d, dst3d)
    e2 = _edge2_call(hs, hd, e1, w["w2s"].astype(bf), w["w2d"].astype(bf),
                     w["w2e"], w["b2"])
    agg2 = _scatter(e2, dst3d)
    o8 = _node2_call(h1, agg2, deg, tan6, tp,
                     w["wn2x"].astype(bf), w["wn2a"], w["bn2"], w["wh"],
                     w["bh"])
    return (o8[:N, 0:3], o8[:N, 3:4])


# speedup vs baseline: 1.3223x; 1.2830x over previous
import functools

import jax
import jax.numpy as jnp
from jax import lax
from jax.experimental import pallas as pl
from jax.experimental.pallas import tpu as pltpu
from jax.experimental.pallas import tpu_sc as plsc

N = 50000
E = 800000
NC = 2
NS = 16
NW = NC * NS
CH = 128
NCHUNK = 6272
EP = NCHUNK * CH
CPW = NCHUNK // NW
CPT = NCHUNK // NS
NP = 50176
TR = NP // NS
ZR = 112
NZ = 28
GW = 16
NG = 8
FW = 48
FWB = 64
LW = 128
BE = 4096
BN = 1792


def _mesh():
    return plsc.VectorSubcoreMesh(core_axis_name="c", subcore_axis_name="s")


def _make_gather(width, dtype):
    out_type = [
        jax.ShapeDtypeStruct((EP, width), dtype),
        jax.ShapeDtypeStruct((EP, width), dtype),
    ]
    scratch = [
        pltpu.VMEM((CPW, CH), jnp.int32),
        pltpu.VMEM((CPW, CH), jnp.int32),
        pltpu.VMEM((2, CH, width), dtype),
        pltpu.VMEM((2, CH, width), dtype),
    ] + [pltpu.SemaphoreType.DMA] * 8

    @functools.partial(
        pl.kernel, mesh=_mesh(), out_type=out_type, scratch_types=scratch,
        compiler_params=pltpu.CompilerParams(use_tc_tiling_on_sc=False))
    def gather_k(table, src3d, dst3d, xs, xd, sidx, didx, sbuf, dbuf, *sems):
        gs = sems[0:2]
        gd = sems[2:4]
        ws = sems[4:6]
        wd = sems[6:8]
        c = lax.axis_index("c")
        s = lax.axis_index("s")
        w = s * NC + c

        pltpu.sync_copy(src3d.at[w], sidx)
        pltpu.sync_copy(dst3d.at[w], didx)

        def g_desc(buf, idx, slot, chunk, sem):
            return pltpu.make_async_copy(
                table.at[idx.at[chunk]], buf.at[slot], sem)

        def w_desc(buf, out, slot, chunk, sem):
            base = (w * CPW + chunk) * CH
            return pltpu.make_async_copy(
                buf.at[slot], out.at[pl.ds(base, CH)], sem)

        for slot in range(2):
            g_desc(sbuf, sidx, slot, slot, gs[slot]).start()
            g_desc(dbuf, didx, slot, slot, gd[slot]).start()

        def step(m, first, last):
            for slot in range(2):
                chunk = 2 * m + slot
                g_desc(sbuf, sidx, slot, chunk, gs[slot]).wait()
                w_desc(sbuf, xs, slot, chunk, ws[slot]).start()
                g_desc(dbuf, didx, slot, chunk, gd[slot]).wait()
                w_desc(dbuf, xd, slot, chunk, wd[slot]).start()
            for slot in range(2):
                chunk = 2 * m + slot
                w_desc(sbuf, xs, slot, chunk, ws[slot]).wait()
                if not last:
                    g_desc(sbuf, sidx, slot, chunk + 2, gs[slot]).start()
                w_desc(dbuf, xd, slot, chunk, wd[slot]).wait()
                if not last:
                    g_desc(dbuf, didx, slot, chunk + 2, gd[slot]).start()
            return 0

        lax.fori_loop(0, CPW // 2 - 1, lambda m, _: step(m, False, False), 0)
        step(CPW // 2 - 1, False, True)

    return gather_k


def _make_deg():
    scratch = [
        pltpu.VMEM((CPW, CH), jnp.int32),
        pltpu.VMEM((CH, 16), jnp.float32),
        pltpu.VMEM((ZR, 16), jnp.float32),
        pltpu.VMEM_SHARED((NP, 16), jnp.float32),
    ]

    @functools.partial(
        pl.kernel, mesh=_mesh(),
        out_type=jax.ShapeDtypeStruct((NC, NP, 16), jnp.float32),
        scratch_types=scratch,
        compiler_params=pltpu.CompilerParams(use_tc_tiling_on_sc=False))
    def deg_k(dst3d, deg_hbm, didx, ones_v, zbuf, deg_sp):
        c = lax.axis_index("c")
        s = lax.axis_index("s")
        w = s * NC + c

        pltpu.sync_copy(dst3d.at[w], didx)

        def fill(i, _):
            ones_v[i] = jnp.ones((16,), jnp.float32)
            return 0
        lax.fori_loop(0, CH, fill, 0)

        def zfill(i, _):
            zbuf[i] = jnp.zeros((16,), jnp.float32)
            return 0
        lax.fori_loop(0, ZR, zfill, 0)
        for k3 in range(NZ):
            pltpu.sync_copy(zbuf, deg_sp.at[pl.ds(s * TR + k3 * ZR, ZR)])
        plsc.subcore_barrier()

        def chunk(j, _):
            pltpu.sync_copy(ones_v, deg_sp.at[didx.at[j]], add=True)
            return 0
        lax.fori_loop(0, CPW, chunk, 0)

        plsc.subcore_barrier()
        for k3 in range(NZ):
            r0 = s * TR + k3 * ZR
            pltpu.sync_copy(deg_sp.at[pl.ds(r0, ZR)],
                            deg_hbm.at[c, pl.ds(r0, ZR)])

    return deg_k


NB = 8


def _make_scatter():
    scratch = [
        pltpu.VMEM((CPT, CH), jnp.int32),
        pltpu.VMEM((NB, CH, GW), jnp.float32),
        pltpu.VMEM((ZR, GW), jnp.float32),
        pltpu.VMEM_SHARED((NP, GW), jnp.float32),
    ] + [pltpu.SemaphoreType.DMA] * (2 * NB)

    @functools.partial(
        pl.kernel, mesh=_mesh(),
        out_type=jax.ShapeDtypeStruct((NP, LW), jnp.float32),
        scratch_types=scratch,
        compiler_params=pltpu.CompilerParams(use_tc_tiling_on_sc=False))
    def scatter_k(e_hbm, dst3d, agg_hbm, didx, ebuf, zbuf, acc_sp, *sems):
        rs = sems[0:NB]
        ss = sems[NB:2 * NB]
        c = lax.axis_index("c")
        s = lax.axis_index("s")

        for h in range(CPT // CPW):
            pltpu.sync_copy(dst3d.at[(CPT // CPW) * s + h],
                            didx.at[pl.ds(h * CPW, CPW)])

        def zfill(i, _):
            zbuf[i] = jnp.zeros((16,), jnp.float32)
            return 0
        lax.fori_loop(0, ZR, zfill, 0)

        for gi in range(NG // NC):
            g = (NG // NC) * c + gi
            for k3 in range(NZ):
                pltpu.sync_copy(
                    zbuf, acc_sp.at[pl.ds(s * TR + k3 * ZR, ZR)])
            plsc.subcore_barrier()

            def r_desc(b, chunk):
                base = (s * CPT + chunk) * CH
                return pltpu.make_async_copy(
                    e_hbm.at[pl.ds(base, CH), pl.ds(g * GW, GW)],
                    ebuf.at[b], rs[b])

            def s_desc(b, chunk):
                return pltpu.make_async_copy(
                    ebuf.at[b], acc_sp.at[didx.at[chunk]], ss[b])

            for b in range(NB):
                r_desc(b, b).start()

            def round_(k, last):
                for b in range(NB):
                    chunk = k * NB + b
                    r_desc(b, chunk).wait()
                    s_desc(b, chunk).start(add=True)
                for b in range(NB):
                    chunk = k * NB + b
                    s_desc(b, chunk).wait()
                    if not last:
                        r_desc(b, chunk + NB).start()
                return 0

            lax.fori_loop(0, CPT // NB - 1,
                          lambda k, _: round_(k, False), 0)
            round_(CPT // NB - 1, True)
            plsc.subcore_barrier()

            for k3 in range(NZ):
                r0 = s * TR + k3 * ZR
                pltpu.sync_copy(acc_sp.at[pl.ds(r0, ZR)],
                                agg_hbm.at[pl.ds(r0, ZR),
                                           pl.ds(g * GW, GW)])
            plsc.subcore_barrier()

    return scatter_k


_DOT = functools.partial(jnp.dot, preferred_element_type=jnp.float32,
                         precision=jax.lax.Precision.DEFAULT)


def _edge1_body(xs, xd, gpbd, wxs, wxd, wg, b1, out):
    m = _DOT(xs[...], wxs[...]) + _DOT(xd[...], wxd[...])
    m = m + _DOT(gpbd[...], wg[...])
    out[...] = jnp.maximum(m + b1[...], 0.0)


def _edge2_body(hs, hd, e1, w2s, w2d, w2e, b2, out):
    m = _DOT(hs[...], w2s[...]) + _DOT(hd[...], w2d[...])
    m = m + _DOT(e1[...], w2e[...])
    out[...] = jnp.maximum(m + b2[...], 0.0)


def _node1_body(feat, agg, deg, wnx, wna, bn, out):
    d = jnp.maximum(deg[0, :, 0:1] + deg[1, :, 0:1], 1.0)
    aggn = agg[...] / d
    h = jnp.maximum(_DOT(feat[...], wnx[...]) + _DOT(aggn, wna[...])
                    + bn[...], 0.0)
    out[...] = h


def _node2_body(h1, agg, deg, tan, t, wnx, wna, bn, wh, bh, out):
    d = jnp.maximum(deg[0, :, 0:1] + deg[1, :, 0:1], 1.0)
    aggn = agg[...] / d
    h2 = jnp.maximum(_DOT(h1[...], wnx[...]) + _DOT(aggn, wna[...])
                     + bn[...], 0.0)
    head = _DOT(h2, wh[...]) + bh[...]
    eh = jnp.exp(-jnp.abs(head[:, 2:3]) / t[...])
    bh_ = jnp.abs(head[:, 3:4])
    sg = jnp.abs(head[:, 4:5])
    o3 = (head[:, 0:1] * tan[:, 0:3] + head[:, 1:2] * tan[:, 3:6]) / bh_ * eh
    out[...] = jnp.concatenate(
        [o3, sg, jnp.zeros((o3.shape[0], 4), jnp.float32)], axis=1)


def _full(shape):
    return pl.BlockSpec(shape, lambda i: tuple(0 for _ in shape))


def _edge1_call(xs, xd, gpbd, wxs, wxd, wg, b1):
    return pl.pallas_call(
        _edge1_body,
        grid=(EP // BE,),
        in_specs=[
            pl.BlockSpec((BE, FW), lambda i: (i, 0)),
            pl.BlockSpec((BE, FW), lambda i: (i, 0)),
            pl.BlockSpec((BE, 32), lambda i: (i, 0)),
            _full((FW, LW)), _full((FW, LW)), _full((32, LW)),
            _full((1, LW)),
        ],
        out_specs=pl.BlockSpec((BE, LW), lambda i: (i, 0)),
        out_shape=jax.ShapeDtypeStruct((EP, LW), jnp.float32),
    )(xs, xd, gpbd, wxs, wxd, wg, b1)


def _edge2_call(hs, hd, e1, w2s, w2d, w2e, b2):
    return pl.pallas_call(
        _edge2_body,
        grid=(EP // BE,),
        in_specs=[
            pl.BlockSpec((BE, LW), lambda i: (i, 0)),
            pl.BlockSpec((BE, LW), lambda i: (i, 0)),
            pl.BlockSpec((BE, LW), lambda i: (i, 0)),
            _full((LW, LW)), _full((LW, LW)), _full((LW, LW)),
            _full((1, LW)),
        ],
        out_specs=pl.BlockSpec((BE, LW), lambda i: (i, 0)),
        out_shape=jax.ShapeDtypeStruct((EP, LW), jnp.float32),
    )(hs, hd, e1, w2s, w2d, w2e, b2)


def _node1_call(feat, agg, deg, wnx, wna, bn):
    return pl.pallas_call(
        _node1_body,
        grid=(NP // BN,),
        in_specs=[
            pl.BlockSpec((BN, FW), lambda i: (i, 0)),
            pl.BlockSpec((BN, LW), lambda i: (i, 0)),
            pl.BlockSpec((2, BN, 16), lambda i: (0, i, 0)),
            _full((FW, LW)), _full((LW, LW)), _full((1, LW)),
        ],
        out_specs=pl.BlockSpec((BN, LW), lambda i: (i, 0)),
        out_shape=jax.ShapeDtypeStruct((NP, LW), jnp.float32),
    )(feat, agg, deg, wnx, wna, bn)


def _node2_call(h1, agg, deg, tan6, t, wnx, wna, bn, wh, bh):
    return pl.pallas_call(
        _node2_body,
        grid=(NP // BN,),
        in_specs=[
            pl.BlockSpec((BN, LW), lambda i: (i, 0)),
            pl.BlockSpec((BN, LW), lambda i: (i, 0)),
            pl.BlockSpec((2, BN, 16), lambda i: (0, i, 0)),
            pl.BlockSpec((BN, 6), lambda i: (i, 0)),
            pl.BlockSpec((BN, 1), lambda i: (i, 0)),
            _full((LW, LW)), _full((LW, LW)), _full((1, LW)),
            _full((LW, 8)), _full((1, 8)),
        ],
        out_specs=pl.BlockSpec((BN, 8), lambda i: (i, 0)),
        out_shape=jax.ShapeDtypeStruct((NP, 8), jnp.float32),
    )(h1, agg, deg, tan6, t, wnx, wna, bn, wh, bh)


def _pack_weights(params):
    z = jnp.zeros
    f32 = jnp.float32
    stacks1 = ["conv1", "Econv1", "Bconv1", "sigconv1"]
    xw = [32, 32, 33, 33]

    w1xs = z((FW, LW), f32)
    w1xd = z((FW, LW), f32)
    w1g = z((32, LW), f32)
    b1 = []
    wn1x = z((FW, LW), f32)
    wn1a = z((LW, LW), f32)
    bn1 = []
    for s4, name in enumerate(stacks1):
        ew = params[name]["edge"]["W"]
        nw = params[name]["node"]["W"]
        k = xw[s4]
        c0 = 32 * s4
        w1xs = w1xs.at[0:k, c0:c0 + 32].set(ew[0:k])
        w1xd = w1xd.at[0:k, c0:c0 + 32].set(ew[k:2 * k])
        if name == "Bconv1":
            w1g = w1g.at[16:17, c0:c0 + 32].set(ew[2 * k:2 * k + 1])
        else:
            w1g = w1g.at[0:16, c0:c0 + 32].set(ew[2 * k:2 * k + 16])
        b1.append(params[name]["edge"]["b"])
        wn1x = wn1x.at[0:k, c0:c0 + 32].set(nw[0:k])
        wn1a = wn1a.at[c0:c0 + 32, c0:c0 + 32].set(nw[k:k + 32])
        bn1.append(params[name]["node"]["b"])
    b1 = jnp.concatenate(b1).reshape(1, LW)
    bn1 = jnp.concatenate(bn1).reshape(1, LW)

    w2s = z((LW, LW), f32)
    w2d = z((LW, LW), f32)
    w2e = z((LW, LW), f32)
    b2 = []
    wn2x = z((LW, LW), f32)
    wn2a = z((LW, LW), f32)
    bn2 = []
    for s4, name in enumerate(["conv2", "Econv2", "Bconv2", "sigconv2"]):
        ew = params[name]["edge"]["W"]
        nw = params[name]["node"]["W"]
        c0 = 32 * s4
        w2s = w2s.at[c0:c0 + 32, c0:c0 + 32].set(ew[0:32])
        w2d = w2d.at[c0:c0 + 32, c0:c0 + 32].set(ew[32:64])
        w2e = w2e.at[c0:c0 + 32, c0:c0 + 32].set(ew[64:96])
        b2.append(params[name]["edge"]["b"])
        wn2x = wn2x.at[c0:c0 + 32, c0:c0 + 32].set(nw[0:32])
        wn2a = wn2a.at[c0:c0 + 32, c0:c0 + 32].set(nw[32:64])
        bn2.append(params[name]["node"]["b"])
    b2 = jnp.concatenate(b2).reshape(1, LW)
    bn2 = jnp.concatenate(bn2).reshape(1, LW)

    wh = z((LW, 8), f32)
    wh = wh.at[0:32, 0:2].set(params["linear"]["W"])
    wh = wh.at[32:64, 2:3].set(params["Elinear"]["W"])
    wh = wh.at[64:96, 3:4].set(params["Blinear"]["W"])
    wh = wh.at[96:128, 4:5].set(params["siglinear"]["W"])
    bh = z((1, 8), f32)
    bh = bh.at[0, 0:2].set(params["linear"]["b"])
    bh = bh.at[0, 2].set(params["Elinear"]["b"][0])
    bh = bh.at[0, 3].set(params["Blinear"]["b"][0])
    bh = bh.at[0, 4].set(params["siglinear"]["b"][0])

    return dict(w1xs=w1xs, w1xd=w1xd, w1g=w1g, b1=b1,
                wn1x=wn1x, wn1a=wn1a, bn1=bn1,
                w2s=w2s, w2d=w2d, w2e=w2e, b2=b2,
                wn2x=wn2x, wn2a=wn2a, bn2=bn2, wh=wh, bh=bh)


_make_gather = functools.cache(_make_gather)
_make_deg = functools.cache(_make_deg)
_make_scatter = functools.cache(_make_scatter)


def kernel(edge_index, feature_GP, feature_Node, feature_bdott, feature_tan,
           feature_T, params):
    f32 = jnp.float32
    src = edge_index[0]
    dst = edge_index[1]
    pad = EP - E
    src2d = jnp.concatenate(
        [src, jnp.zeros((pad,), jnp.int32)]).reshape(NCHUNK, CH)
    dst2d = jnp.concatenate(
        [dst, jnp.full((pad,), N, jnp.int32)]).reshape(NCHUNK, CH)
    featp = jnp.concatenate(
        [feature_Node, feature_T, jnp.zeros((N, FW - 33), f32)], axis=1)
    featp = jnp.concatenate([featp, jnp.zeros((NP - N, FW), f32)], axis=0)
    gpbd = jnp.concatenate(
        [feature_GP, feature_bdott, jnp.zeros((E, 15), f32)], axis=1)
    gpbd = jnp.concatenate([gpbd, jnp.zeros((pad, 32), f32)], axis=0)
    tan6 = jnp.concatenate(
        [feature_tan.reshape(N, 6), jnp.zeros((NP - N, 6), f32)], axis=0)
    tp = jnp.concatenate([feature_T, jnp.ones((NP - N, 1), f32)], axis=0)
    w = _pack_weights(params)


    _gather48 = _make_gather(FW, jnp.float32)
    _gather128 = _make_gather(LW, jnp.float32)
    _scatter = _make_scatter()

    src3d = src2d.reshape(NW, CPW, CH)
    dst3d = dst2d.reshape(NW, CPW, CH)

    xs, xd = _gather48(featp, src3d, dst3d)
    deg = _make_deg()(dst3d)
    e1 = _edge1_call(xs, xd, gpbd, w["w1xs"], w["w1xd"], w["w1g"], w["b1"])
    agg1 = _scatter(e1, dst3d)
    h1 = _node1_call(featp, agg1, deg, w["wn1x"], w["wn1a"], w["bn1"])
    hs, hd = _gather128(h1, src3d, dst3d)
    e2 = _edge2_call(hs, hd, e1, w["w2s"], w["w2d"], w["w2e"], w["b2"])
    agg2 = _scatter(e2, dst3d)
    o8 = _node2_call(h1, agg2, deg, tan6, tp,
                     w["wn2x"], w["wn2a"], w["bn2"], w["wh"], w["bh"])
    return (o8[:N, 0:3], o8[:N, 3:4])
